# baseline (speedup 1.0000x reference)
import functools

import jax
import jax.numpy as jnp
from jax import lax
from jax.experimental import pallas as pl
from jax.experimental.pallas import tpu as pltpu
from jax.experimental.pallas import tpu_sc as plsc

_NUM_EMB = 90000
_EMB_DIM = 32
_BATCH = 16384
_RPL = 128 // _EMB_DIM
_NLINES = _NUM_EMB // _RPL

_info = plsc.get_sparse_core_info()
_NC = _info.num_cores
_NS = _info.num_subcores
_L = _info.num_lanes
_NW = _NC * _NS
_BPW = _BATCH // _NW
_ICHUNK = 128
_NICHUNK = _BPW // _ICHUNK
_NGRP = _ICHUNK // _L


def _cosine_body(uid_hbm, iid_hbm, ut_hbm, it_hbm, out_hbm,
                 uidx_v, iidx_v, ugl_v, igl_v, ubuf_v, ibuf_v, out_v,
                 sem_idx, sem_g0, sem_g1):
    wid = lax.axis_index("s") * _NC + lax.axis_index("c")
    base = wid * _BPW

    idx_copies = []
    for j in range(_NICHUNK):
        idx_copies.append(pltpu.async_copy(
            uid_hbm.at[pl.ds(base + j * _ICHUNK, _ICHUNK)],
            uidx_v.at[j], sem_idx))
        idx_copies.append(pltpu.async_copy(
            iid_hbm.at[pl.ds(base + j * _ICHUNK, _ICHUNK)],
            iidx_v.at[j], sem_idx))
    for c in idx_copies:
        c.wait()

    for j in range(_NICHUNK):
        for g in range(_NGRP):
            s = pl.ds(g * _L, _L)
            ugl_v.at[j][s] = lax.shift_right_logical(uidx_v.at[j][s], _RPL // 2)
            igl_v.at[j][s] = lax.shift_right_logical(iidx_v.at[j][s], _RPL // 2)

    sems = (sem_g0, sem_g1)

    def fire(j):
        buf = j % 2
        return (pltpu.async_copy(ut_hbm.at[ugl_v.at[j]], ubuf_v.at[buf],
                                 sems[buf]),
                pltpu.async_copy(it_hbm.at[igl_v.at[j]], ibuf_v.at[buf],
                                 sems[buf]))

    def compute_chunk(j):
        buf = j % 2
        ub = ubuf_v.at[buf]
        ib = ibuf_v.at[buf]

        def group_body(g, carry):
            rows = lax.iota(jnp.int32, _L) + g * _L
            s = pl.dslice(g * _L, _L)
            ubase = (uidx_v.at[j][s] & (_RPL - 1)) * _EMB_DIM
            ibase = (iidx_v.at[j][s] & (_RPL - 1)) * _EMB_DIM
            acc_d = jnp.zeros((_L,), jnp.float32)
            acc_u = jnp.zeros((_L,), jnp.float32)
            acc_i = jnp.zeros((_L,), jnp.float32)
            for d in range(_EMB_DIM):
                uc = plsc.load_gather(ub, [rows, ubase + d])
                ic = plsc.load_gather(ib, [rows, ibase + d])
                acc_d = acc_d + uc * ic
                acc_u = acc_u + uc * uc
                acc_i = acc_i + ic * ic
            p = jnp.maximum(acc_u, 1e-16) * jnp.maximum(acc_i, 1e-16)
            seed = jnp.full((_L,), 0x5F3759DF, jnp.int32) - \
                lax.shift_right_logical(plsc.bitcast(p, jnp.int32), 1)
            y = plsc.bitcast(seed, jnp.float32)
            for _ in range(3):
                y = y * (1.5 - 0.5 * p * y * y)
            out_v[pl.ds(j * _ICHUNK + g * _L, _L)] = (6.0 * acc_d) * y
            return carry

        lax.fori_loop(0, _NGRP, group_body, 0)

    inflight = fire(0)
    for j in range(_NICHUNK):
        if j + 1 < _NICHUNK:
            nxt = fire(j + 1)
        for c in inflight:
            c.wait()
        compute_chunk(j)
        if j + 1 < _NICHUNK:
            inflight = nxt

    pltpu.sync_copy(out_v, out_hbm.at[pl.ds(base, _BPW)])


@functools.partial(
    pl.kernel,
    mesh=plsc.VectorSubcoreMesh(core_axis_name="c", subcore_axis_name="s"),
    out_type=jax.ShapeDtypeStruct((_BATCH,), jnp.float32),
    scratch_types=[
        pltpu.VMEM((_NICHUNK, _ICHUNK), jnp.int32),
        pltpu.VMEM((_NICHUNK, _ICHUNK), jnp.int32),
        pltpu.VMEM((_NICHUNK, _ICHUNK), jnp.int32),
        pltpu.VMEM((_NICHUNK, _ICHUNK), jnp.int32),
        pltpu.VMEM((2, _ICHUNK, 128), jnp.float32),
        pltpu.VMEM((2, _ICHUNK, 128), jnp.float32),
        pltpu.VMEM((_BPW,), jnp.float32),
        pltpu.SemaphoreType.DMA,
        pltpu.SemaphoreType.DMA,
        pltpu.SemaphoreType.DMA,
    ],
    compiler_params=pltpu.CompilerParams(needs_layout_passes=False,
                                         use_tc_tiling_on_sc=True),
)
def _cosine_sc(uid_hbm, iid_hbm, ut_hbm, it_hbm, out_hbm,
               uidx_v, iidx_v, ugl_v, igl_v, ubuf_v, ibuf_v, out_v,
               sem_idx, sem_g0, sem_g1):
    _cosine_body(uid_hbm, iid_hbm, ut_hbm, it_hbm, out_hbm,
                 uidx_v, iidx_v, ugl_v, igl_v, ubuf_v, ibuf_v, out_v,
                 sem_idx, sem_g0, sem_g1)


def kernel(user_id, item_id, user_table, item_table):
    return _cosine_sc(user_id.astype(jnp.int32), item_id.astype(jnp.int32),
                      user_table.reshape(_NLINES, 128),
                      item_table.reshape(_NLINES, 128))


# speedup vs baseline: 1.3487x; 1.3487x over previous
import functools

import jax
import jax.numpy as jnp
from jax import lax
from jax.experimental import pallas as pl
from jax.experimental.pallas import tpu as pltpu
from jax.experimental.pallas import tpu_sc as plsc

_NUM_EMB = 90000
_EMB_DIM = 32
_BATCH = 16384
_FLAT = _NUM_EMB * _EMB_DIM

_info = plsc.get_sparse_core_info()
_NC = _info.num_cores
_NS = _info.num_subcores
_L = _info.num_lanes
_NW = _NC * _NS
_BPW = _BATCH // _NW
_ICHUNK = 128
_NICHUNK = _BPW // _ICHUNK
_NGRP = _ICHUNK // _L


def _cosine_body(uid_hbm, iid_hbm, ut_hbm, it_hbm, out_hbm,
                 uidx_v, iidx_v, ubuf_v, ibuf_v, out_v,
                 sem_idx, sem_u, sem_i):
    wid = lax.axis_index("s") * _NC + lax.axis_index("c")
    base = wid * _BPW

    idx_copies = []
    for j in range(_NICHUNK):
        idx_copies.append(pltpu.async_copy(
            uid_hbm.at[pl.ds(base + j * _ICHUNK, _ICHUNK)],
            uidx_v.at[j], sem_idx))
        idx_copies.append(pltpu.async_copy(
            iid_hbm.at[pl.ds(base + j * _ICHUNK, _ICHUNK)],
            iidx_v.at[j], sem_idx))
    for c in idx_copies:
        c.wait()

    def fire(j):
        buf = j % 2

        def fire_dim(d, carry):
            src_u = ut_hbm.at[pl.ds(d * _NUM_EMB, _NUM_EMB)]
            src_i = it_hbm.at[pl.ds(d * _NUM_EMB, _NUM_EMB)]
            off = buf * (_EMB_DIM * _ICHUNK) + d * _ICHUNK
            pltpu.async_copy(src_u.at[uidx_v.at[j]],
                             ubuf_v.at[pl.ds(off, _ICHUNK)], sem_u)
            pltpu.async_copy(src_i.at[iidx_v.at[j]],
                             ibuf_v.at[pl.ds(off, _ICHUNK)], sem_i)
            return carry

        lax.fori_loop(0, _EMB_DIM, fire_dim, 0)

    def drain(j):
        buf = j % 2
        half = _EMB_DIM * _ICHUNK
        pltpu.make_async_copy(ut_hbm.at[pl.ds(0, half)],
                              ubuf_v.at[pl.ds(buf * half, half)], sem_u).wait()
        pltpu.make_async_copy(it_hbm.at[pl.ds(0, half)],
                              ibuf_v.at[pl.ds(buf * half, half)], sem_i).wait()

    def compute_chunk(j):
        buf = j % 2

        def group_body(g, carry):
            acc_d = jnp.zeros((_L,), jnp.float32)
            acc_u = jnp.zeros((_L,), jnp.float32)
            acc_i = jnp.zeros((_L,), jnp.float32)
            for d in range(_EMB_DIM):
                s = pl.ds(buf * (_EMB_DIM * _ICHUNK) + d * _ICHUNK + g * _L,
                          _L)
                uc = ubuf_v[s]
                ic = ibuf_v[s]
                acc_d = acc_d + uc * ic
                acc_u = acc_u + uc * uc
                acc_i = acc_i + ic * ic
            p = jnp.maximum(acc_u, 1e-16) * jnp.maximum(acc_i, 1e-16)
            seed = jnp.full((_L,), 0x5F3759DF, jnp.int32) - \
                lax.shift_right_logical(plsc.bitcast(p, jnp.int32), 1)
            y = plsc.bitcast(seed, jnp.float32)
            for _ in range(3):
                y = y * (1.5 - 0.5 * p * y * y)
            out_v[pl.ds(j * _ICHUNK + g * _L, _L)] = (6.0 * acc_d) * y
            return carry

        lax.fori_loop(0, _NGRP, group_body, 0)

    fire(0)
    for j in range(_NICHUNK):
        if j + 1 < _NICHUNK:
            fire(j + 1)
        drain(j)
        compute_chunk(j)

    pltpu.sync_copy(out_v, out_hbm.at[pl.ds(base, _BPW)])


@functools.partial(
    pl.kernel,
    mesh=plsc.VectorSubcoreMesh(core_axis_name="c", subcore_axis_name="s"),
    out_type=jax.ShapeDtypeStruct((_BATCH,), jnp.float32),
    scratch_types=[
        pltpu.VMEM((_NICHUNK, _ICHUNK), jnp.int32),
        pltpu.VMEM((_NICHUNK, _ICHUNK), jnp.int32),
        pltpu.VMEM((2 * _EMB_DIM * _ICHUNK,), jnp.float32),
        pltpu.VMEM((2 * _EMB_DIM * _ICHUNK,), jnp.float32),
        pltpu.VMEM((_BPW,), jnp.float32),
        pltpu.SemaphoreType.DMA,
        pltpu.SemaphoreType.DMA,
        pltpu.SemaphoreType.DMA,
    ],
    compiler_params=pltpu.CompilerParams(needs_layout_passes=False),
)
def _cosine_sc(uid_hbm, iid_hbm, ut_hbm, it_hbm, out_hbm,
               uidx_v, iidx_v, ubuf_v, ibuf_v, out_v,
               sem_idx, sem_u, sem_i):
    _cosine_body(uid_hbm, iid_hbm, ut_hbm, it_hbm, out_hbm,
                 uidx_v, iidx_v, ubuf_v, ibuf_v, out_v,
                 sem_idx, sem_u, sem_i)


def kernel(user_id, item_id, user_table, item_table):
    return _cosine_sc(user_id.astype(jnp.int32), item_id.astype(jnp.int32),
                      user_table.T.reshape(_FLAT),
                      item_table.T.reshape(_FLAT))
